# Initial kernel scaffold; baseline (speedup 1.0000x reference)
import jax
import jax.numpy as jnp
from jax.experimental import pallas as pl


def kernel(x, edge_index, edge_attr, W, b, W1, b1, W2, b2):
    raise NotImplementedError("write your pallas kernel here")


# speedup vs baseline: 4.5763x; 4.5763x over previous
import functools

import jax
import jax.numpy as jnp
from jax import lax
from jax.experimental import pallas as pl
from jax.experimental.pallas import tpu as pltpu
from jax.experimental.pallas import tpu_sc as plsc

NC = 2
NS = 16
NW = NC * NS
L = 16


def _matmul_body(x_ref, w_ref, b_ref, o_ref):
    o_ref[...] = lax.dot_general(
        x_ref[...], w_ref[...],
        (((1,), (1,)), ((), ())),
        preferred_element_type=jnp.float32,
    ) + b_ref[...][None, :]


def _linear(x, W, b, blk=2000):
    n, d = x.shape
    grid = n // blk
    return pl.pallas_call(
        _matmul_body,
        grid=(grid,),
        in_specs=[
            pl.BlockSpec((blk, d), lambda i: (i, 0)),
            pl.BlockSpec((d, d), lambda i: (0, 0)),
            pl.BlockSpec((d,), lambda i: (0,)),
        ],
        out_specs=pl.BlockSpec((blk, d), lambda i: (i, 0)),
        out_shape=jax.ShapeDtypeStruct((n, d), jnp.float32),
    )(x, W, b)


def _add_body(a_ref, b_ref, o_ref):
    o_ref[...] = a_ref[...] + b_ref[...]


def _combine(p, blk=2000):
    _, n, d = p.shape
    grid = n // blk
    return pl.pallas_call(
        _add_body,
        grid=(grid,),
        in_specs=[
            pl.BlockSpec((blk, d), lambda i: (i, 0)),
            pl.BlockSpec((blk, d), lambda i: (i, 0)),
        ],
        out_specs=pl.BlockSpec((blk, d), lambda i: (i, 0)),
        out_shape=jax.ShapeDtypeStruct((n, d), jnp.float32),
    )(p[0], p[1])


C = 80
ZROWS = 160


def _make_agg(n, d, e):
    epw = e // NW
    chunks = epw // C
    rows_per_tile = -(-n // (NS * ZROWS)) * ZROWS
    n_pad = rows_per_tile * NS
    last_rows = n - rows_per_tile * (NS - 1)
    assert rows_per_tile % ZROWS == 0 and last_rows % 8 == 0

    @functools.partial(
        pl.kernel,
        out_type=jax.ShapeDtypeStruct((NC, n, d), jnp.float32),
        mesh=plsc.VectorSubcoreMesh(core_axis_name="c", subcore_axis_name="s"),
        scratch_types=[
            pltpu.VMEM((C,), jnp.int32),
            pltpu.VMEM((C,), jnp.int32),
            pltpu.VMEM((C,), jnp.float32),
            pltpu.VMEM((C, d), jnp.float32),
            pltpu.VMEM((ZROWS, d), jnp.float32),
            pltpu.VMEM_SHARED((n_pad, d), jnp.float32),
            pltpu.SemaphoreType.DMA,
        ],
    )
    def agg(xl_hbm, col_hbm, row_hbm, attr_hbm, out_hbm,
            colv, rowv, attrv, rowsv, zv, acc, sem):
        cid = lax.axis_index("c")
        sid = lax.axis_index("s")
        wid = sid * NC + cid

        def zfill(i, _):
            zero = jnp.zeros((L,), jnp.float32)
            for k in range(d // L):
                zv[i, pl.ds(k * L, L)] = zero
            return 0
        lax.fori_loop(0, ZROWS, zfill, 0)
        for j in range(rows_per_tile // ZROWS):
            zoff = pl.multiple_of(sid * rows_per_tile + j * ZROWS, 8)
            pltpu.sync_copy(zv, acc.at[pl.ds(zoff, ZROWS)])
        plsc.subcore_barrier()

        def chunk(g, _):
            off = pl.multiple_of(wid * epw + g * C, 8)
            pltpu.sync_copy(col_hbm.at[pl.ds(off, C)], colv)
            pltpu.sync_copy(attr_hbm.at[pl.ds(off, C)], attrv)
            pltpu.sync_copy(row_hbm.at[pl.ds(off, C)], rowv)
            pltpu.async_copy(xl_hbm.at[colv], rowsv, sem).wait()

            def scale(g16, _):
                av = attrv[pl.ds(g16 * L, L)]
                for j in range(L):
                    a = av[j]
                    base = g16 * L + j
                    for k in range(d // L):
                        sl = pl.ds(k * L, L)
                        rowsv[base, sl] = rowsv[base, sl] * a
                return 0
            lax.fori_loop(0, C // L, scale, 0)

            pltpu.sync_copy(rowsv, acc.at[rowv], add=True)
            return 0
        lax.fori_loop(0, chunks, chunk, 0)

        plsc.subcore_barrier()
        woff = pl.multiple_of(sid * rows_per_tile, 8)

        @pl.when(sid < NS - 1)
        def _():
            pltpu.sync_copy(
                acc.at[pl.ds(woff, rows_per_tile)],
                out_hbm.at[cid, pl.ds(woff, rows_per_tile)],
            )

        @pl.when(sid == NS - 1)
        def _():
            loff = pl.multiple_of((NS - 1) * rows_per_tile, 8)
            pltpu.sync_copy(
                acc.at[pl.ds(loff, last_rows)],
                out_hbm.at[cid, pl.ds(loff, last_rows)],
            )

    return agg


def kernel(x, edge_index, edge_attr, W, b, W1, b1, W2, b2):
    n, d = x.shape
    e = edge_attr.shape[0]
    row = edge_index[0].astype(jnp.int32)
    col = edge_index[1].astype(jnp.int32)
    xl = _linear(x, W, b)
    partials = _make_agg(n, d, e)(xl, col, row, edge_attr, )
    return _combine(partials)


# speedup vs baseline: 10.8055x; 2.3612x over previous
import functools

import jax
import jax.numpy as jnp
from jax import lax
from jax.experimental import pallas as pl
from jax.experimental.pallas import tpu as pltpu
from jax.experimental.pallas import tpu_sc as plsc

NC = 2
NS = 16
NW = NC * NS
L = 16


def _matmul_body(x_ref, w_ref, b_ref, o_ref):
    o_ref[...] = lax.dot_general(
        x_ref[...], w_ref[...],
        (((1,), (1,)), ((), ())),
        preferred_element_type=jnp.float32,
    ) + b_ref[...][None, :]


def _linear(x, W, b, blk=2000):
    n, d = x.shape
    grid = n // blk
    return pl.pallas_call(
        _matmul_body,
        grid=(grid,),
        in_specs=[
            pl.BlockSpec((blk, d), lambda i: (i, 0)),
            pl.BlockSpec((d, d), lambda i: (0, 0)),
            pl.BlockSpec((d,), lambda i: (0,)),
        ],
        out_specs=pl.BlockSpec((blk, d), lambda i: (i, 0)),
        out_shape=jax.ShapeDtypeStruct((n, d), jnp.float32),
    )(x, W, b)


def _add_body(a_ref, b_ref, o_ref):
    o_ref[...] = a_ref[...] + b_ref[...]


def _combine(p, blk=2000):
    _, n, d = p.shape
    grid = n // blk
    return pl.pallas_call(
        _add_body,
        grid=(grid,),
        in_specs=[
            pl.BlockSpec((blk, d), lambda i: (i, 0)),
            pl.BlockSpec((blk, d), lambda i: (i, 0)),
        ],
        out_specs=pl.BlockSpec((blk, d), lambda i: (i, 0)),
        out_shape=jax.ShapeDtypeStruct((n, d), jnp.float32),
    )(p[0], p[1])


C = 80
NBUF = 4
ZCOPIES = 8


def _make_agg(n, d, e):
    epw = e // NW
    chunks = epw // C
    tail = chunks % NBUF
    lc = chunks - tail
    assert lc >= 2 * NBUF
    rows_per_tile = -(-n // (NS * C * ZCOPIES)) * C * ZCOPIES
    n_pad = rows_per_tile * NS
    last_rows = n - rows_per_tile * (NS - 1)
    assert last_rows % 8 == 0 and last_rows > 0

    @functools.partial(
        pl.kernel,
        out_type=jax.ShapeDtypeStruct((NC, n, d), jnp.float32),
        mesh=plsc.VectorSubcoreMesh(core_axis_name="c", subcore_axis_name="s"),
        scratch_types=[
            [pltpu.VMEM((3, C), jnp.int32) for _ in range(NBUF)],
            [pltpu.VMEM((C,), jnp.int32) for _ in range(NBUF)],
            [pltpu.VMEM((C, d), jnp.float32) for _ in range(NBUF)],
            pltpu.VMEM_SHARED((n_pad, d), jnp.float32),
            [pltpu.SemaphoreType.DMA for _ in range(NBUF)],
            [pltpu.SemaphoreType.DMA for _ in range(NBUF)],
            [pltpu.SemaphoreType.DMA for _ in range(NBUF)],
        ],
    )
    def agg(xl_hbm, meta_hbm, out_hbm,
            meta, rowi, ring, acc, msem, gsem, ssem):
        cid = lax.axis_index("c")
        sid = lax.axis_index("s")
        wid = sid * NC + cid

        def zfill(i, _):
            zero = jnp.zeros((L,), jnp.float32)
            for k in range(d // L):
                ring[0][i, pl.ds(k * L, L)] = zero
            return 0
        lax.fori_loop(0, C, zfill, 0)
        for j in range(ZCOPIES):
            zoff = pl.multiple_of(sid * rows_per_tile + j * C, 8)
            pltpu.sync_copy(ring[0], acc.at[pl.ds(zoff, C)])
        plsc.subcore_barrier()

        def meta_load(g, b):
            pltpu.async_copy(meta_hbm.at[wid, g], meta[b], msem[b])

        def meta_wait(b):
            pltpu.make_async_copy(meta_hbm.at[wid, 0], meta[b], msem[b]).wait()

        def gather(g, b):
            pltpu.async_copy(xl_hbm.at[meta[b].at[0]], ring[b], gsem[b])

        def gather_wait(b):
            pltpu.make_async_copy(xl_hbm.at[meta[b].at[0]], ring[b],
                                  gsem[b]).wait()

        def scatter(b):
            pltpu.async_copy(ring[b], acc.at[rowi[b]], ssem[b], add=True)

        def scatter_wait(b):
            pltpu.make_async_copy(ring[b], acc.at[rowi[b]], ssem[b]).wait()

        def process(g, b):
            for k in range(C // L):
                rowi[b][pl.ds(k * L, L)] = meta[b][1, pl.ds(k * L, L)]

            def scale(j16, _):
                av = lax.bitcast_convert_type(
                    meta[b][2, pl.ds(j16 * L, L)], jnp.float32)
                for j in range(L):
                    a = av[j]
                    base = j16 * L + j
                    for k in range(d // L):
                        sl = pl.ds(k * L, L)
                        ring[b][base, sl] = ring[b][base, sl] * a
                return 0
            lax.fori_loop(0, C // L, scale, 0)

        for b in range(NBUF):
            meta_load(b, b)
        for b in range(2):
            meta_wait(b)
            gather(b, b)

        def outer(o, _):
            for b in range(NBUF):
                g = o * NBUF + b

                @pl.when(g + 2 < lc)
                def _():
                    meta_wait((b + 2) % NBUF)

                @pl.when(g >= 2)
                def _():
                    scatter_wait((b + 2) % NBUF)

                @pl.when(g + 2 < lc)
                def _():
                    gather(g + 2, (b + 2) % NBUF)

                gather_wait(b)
                process(g, b)
                scatter(b)

                @pl.when(g + NBUF < chunks)
                def _():
                    meta_load(g + NBUF, b)
            return 0
        lax.fori_loop(0, lc // NBUF, outer, 0)

        scatter_wait((lc - 2) % NBUF)
        scatter_wait((lc - 1) % NBUF)

        for g in range(lc, chunks):
            b = g % NBUF
            meta_wait(b)
            gather(g, b)
            gather_wait(b)
            process(g, b)
            scatter(b)
            scatter_wait(b)

        plsc.subcore_barrier()
        woff = pl.multiple_of(sid * rows_per_tile, 8)

        @pl.when(sid < NS - 1)
        def _():
            pltpu.sync_copy(
                acc.at[pl.ds(woff, rows_per_tile)],
                out_hbm.at[cid, pl.ds(woff, rows_per_tile)],
            )

        @pl.when(sid == NS - 1)
        def _():
            loff = pl.multiple_of((NS - 1) * rows_per_tile, 8)
            pltpu.sync_copy(
                acc.at[pl.ds(loff, last_rows)],
                out_hbm.at[cid, pl.ds(loff, last_rows)],
            )

    return agg


def kernel(x, edge_index, edge_attr, W, b, W1, b1, W2, b2):
    n, d = x.shape
    e = edge_attr.shape[0]
    epw = e // NW
    chunks = epw // C
    row = edge_index[0].astype(jnp.int32).reshape(NW, chunks, C)
    col = edge_index[1].astype(jnp.int32).reshape(NW, chunks, C)
    attr_bits = lax.bitcast_convert_type(
        edge_attr.astype(jnp.float32), jnp.int32).reshape(NW, chunks, C)
    meta = jnp.stack([col, row, attr_bits], axis=2)
    xl = _linear(x, W, b)
    partials = _make_agg(n, d, e)(xl, meta)
    return _combine(partials)


# speedup vs baseline: 11.0430x; 1.0220x over previous
import functools

import jax
import jax.numpy as jnp
from jax import lax
from jax.experimental import pallas as pl
from jax.experimental.pallas import tpu as pltpu
from jax.experimental.pallas import tpu_sc as plsc

NC = 2
NS = 16
NW = NC * NS
L = 16

C = 80
NBUF = 4
ZCOPIES = 8


def _make_agg(n, d, e):
    epw = e // NW
    chunks = epw // C
    tail = chunks % NBUF
    lc = chunks - tail
    assert lc >= 2 * NBUF
    rows_per_tile = -(-n // (NS * C * ZCOPIES)) * C * ZCOPIES
    n_pad = rows_per_tile * NS

    @functools.partial(
        pl.kernel,
        out_type=(
            jax.ShapeDtypeStruct((NC, n_pad, d), jnp.float32),
            jax.ShapeDtypeStruct((NC, n_pad), jnp.float32),
        ),
        mesh=plsc.VectorSubcoreMesh(core_axis_name="c", subcore_axis_name="s"),
        scratch_types=[
            [pltpu.VMEM((3, C), jnp.int32) for _ in range(NBUF)],
            [pltpu.VMEM((C,), jnp.int32) for _ in range(NBUF)],
            [pltpu.VMEM((C,), jnp.float32) for _ in range(NBUF)],
            [pltpu.VMEM((C, d), jnp.float32) for _ in range(NBUF)],
            pltpu.VMEM_SHARED((n_pad, d), jnp.float32),
            pltpu.VMEM_SHARED((n_pad,), jnp.float32),
            [pltpu.SemaphoreType.DMA for _ in range(NBUF)],
            [pltpu.SemaphoreType.DMA for _ in range(NBUF)],
            [pltpu.SemaphoreType.DMA for _ in range(NBUF)],
            [pltpu.SemaphoreType.DMA for _ in range(NBUF)],
        ],
    )
    def agg(x_hbm, meta_hbm, part_hbm, degp_hbm,
            meta, rowi, attrf, ring, acc, dacc, msem, gsem, ssem, dsem):
        cid = lax.axis_index("c")
        sid = lax.axis_index("s")
        wid = sid * NC + cid

        def zfill(i, _):
            zero = jnp.zeros((L,), jnp.float32)
            for k in range(d // L):
                ring[0][i, pl.ds(k * L, L)] = zero
            return 0
        lax.fori_loop(0, C, zfill, 0)
        for k in range(C // L):
            attrf[0][pl.ds(k * L, L)] = jnp.zeros((L,), jnp.float32)
        for j in range(ZCOPIES):
            zoff = pl.multiple_of(sid * rows_per_tile + j * C, 8)
            pltpu.sync_copy(ring[0], acc.at[pl.ds(zoff, C)])
            pltpu.sync_copy(attrf[0], dacc.at[pl.ds(zoff, C)])
        plsc.subcore_barrier()

        def meta_load(g, b):
            pltpu.async_copy(meta_hbm.at[wid, g], meta[b], msem[b])

        def meta_wait(b):
            pltpu.make_async_copy(meta_hbm.at[wid, 0], meta[b], msem[b]).wait()

        def gather(g, b):
            pltpu.async_copy(x_hbm.at[meta[b].at[0]], ring[b], gsem[b])

        def gather_wait(b):
            pltpu.make_async_copy(x_hbm.at[meta[b].at[0]], ring[b],
                                  gsem[b]).wait()

        def scatter(b):
            pltpu.async_copy(ring[b], acc.at[rowi[b]], ssem[b], add=True)
            pltpu.async_copy(attrf[b], dacc.at[rowi[b]], dsem[b], add=True)

        def scatter_wait(b):
            pltpu.make_async_copy(ring[b], acc.at[rowi[b]], ssem[b]).wait()
            pltpu.make_async_copy(attrf[b], dacc.at[rowi[b]], dsem[b]).wait()

        def process(g, b):
            for k in range(C // L):
                rowi[b][pl.ds(k * L, L)] = meta[b][1, pl.ds(k * L, L)]

            def scale(j16, _):
                av = lax.bitcast_convert_type(
                    meta[b][2, pl.ds(j16 * L, L)], jnp.float32)
                attrf[b][pl.ds(j16 * L, L)] = av
                for j in range(L):
                    a = av[j]
                    base = j16 * L + j
                    for k in range(d // L):
                        sl = pl.ds(k * L, L)
                        ring[b][base, sl] = ring[b][base, sl] * a
                return 0
            lax.fori_loop(0, C // L, scale, 0)

        for b in range(NBUF):
            meta_load(b, b)
        for b in range(2):
            meta_wait(b)
            gather(b, b)

        def outer(o, _):
            for b in range(NBUF):
                g = o * NBUF + b

                @pl.when(g + 2 < lc)
                def _():
                    meta_wait((b + 2) % NBUF)

                @pl.when(g >= 2)
                def _():
                    scatter_wait((b + 2) % NBUF)

                @pl.when(g + 2 < lc)
                def _():
                    gather(g + 2, (b + 2) % NBUF)

                gather_wait(b)
                process(g, b)
                scatter(b)

                @pl.when(g + NBUF < chunks)
                def _():
                    meta_load(g + NBUF, b)
            return 0
        lax.fori_loop(0, lc // NBUF, outer, 0)

        scatter_wait((lc - 2) % NBUF)
        scatter_wait((lc - 1) % NBUF)

        for g in range(lc, chunks):
            b = g % NBUF
            meta_wait(b)
            gather(g, b)
            gather_wait(b)
            process(g, b)
            scatter(b)
            scatter_wait(b)

        plsc.subcore_barrier()
        woff = pl.multiple_of(sid * rows_per_tile, 8)
        pltpu.sync_copy(
            acc.at[pl.ds(woff, rows_per_tile)],
            part_hbm.at[cid, pl.ds(woff, rows_per_tile)],
        )
        pltpu.sync_copy(
            dacc.at[pl.ds(woff, rows_per_tile)],
            degp_hbm.at[cid, pl.ds(woff, rows_per_tile)],
        )

    return agg


def _finish_body(p_ref, dg_ref, w_ref, b_ref, o_ref):
    xp = p_ref[0] + p_ref[1]
    acc = lax.dot_general(
        xp, w_ref[...], (((1,), (1,)), ((), ())),
        preferred_element_type=jnp.float32,
    )
    dgs = dg_ref[0] + dg_ref[1]
    o_ref[...] = acc + dgs[:, None] * b_ref[...][None, :]


def _finish(p, dg, W, b, blk=2048):
    _, n_pad, d = p.shape
    grid = n_pad // blk
    return pl.pallas_call(
        _finish_body,
        grid=(grid,),
        in_specs=[
            pl.BlockSpec((NC, blk, d), lambda i: (0, i, 0)),
            pl.BlockSpec((NC, blk), lambda i: (0, i)),
            pl.BlockSpec((d, d), lambda i: (0, 0)),
            pl.BlockSpec((d,), lambda i: (0,)),
        ],
        out_specs=pl.BlockSpec((blk, d), lambda i: (i, 0)),
        out_shape=jax.ShapeDtypeStruct((n_pad, d), jnp.float32),
    )(p, dg, W, b)


def kernel(x, edge_index, edge_attr, W, b, W1, b1, W2, b2):
    n, d = x.shape
    e = edge_attr.shape[0]
    epw = e // NW
    chunks = epw // C
    row = edge_index[0].astype(jnp.int32).reshape(NW, chunks, C)
    col = edge_index[1].astype(jnp.int32).reshape(NW, chunks, C)
    attr_bits = lax.bitcast_convert_type(
        edge_attr.astype(jnp.float32), jnp.int32).reshape(NW, chunks, C)
    meta = jnp.stack([col, row, attr_bits], axis=2)
    partials, degp = _make_agg(n, d, e)(x.astype(jnp.float32), meta)
    return _finish(partials, degp, W, b)[:n]


# speedup vs baseline: 11.7868x; 1.0674x over previous
import functools

import jax
import jax.numpy as jnp
from jax import lax
from jax.experimental import pallas as pl
from jax.experimental.pallas import tpu as pltpu
from jax.experimental.pallas import tpu_sc as plsc

NC = 2
NS = 16
NW = NC * NS
L = 16

C = 80
NBUF = 4
ZCOPIES = 8


def _make_agg(n, d, e):
    epw = e // NW
    chunks = epw // C
    IB = 2 * NBUF
    lc = chunks - chunks % IB
    assert lc >= 2 * IB
    rows_per_tile = -(-n // (NS * C * ZCOPIES)) * C * ZCOPIES
    n_pad = rows_per_tile * NS

    @functools.partial(
        pl.kernel,
        out_type=(
            jax.ShapeDtypeStruct((NC, n_pad, d), jnp.float32),
            jax.ShapeDtypeStruct((NC, n_pad), jnp.float32),
        ),
        mesh=plsc.VectorSubcoreMesh(core_axis_name="c", subcore_axis_name="s"),
        scratch_types=[
            [pltpu.VMEM((1, C), jnp.int32) for _ in range(IB)],
            [pltpu.VMEM((1, C), jnp.int32) for _ in range(IB)],
            [pltpu.VMEM((1, C), jnp.float32) for _ in range(IB)],
            [pltpu.VMEM((C, d), jnp.float32) for _ in range(NBUF)],
            pltpu.VMEM_SHARED((n_pad, d), jnp.float32),
            pltpu.VMEM_SHARED((n_pad,), jnp.float32),
            [pltpu.SemaphoreType.DMA for _ in range(IB)],
            [pltpu.SemaphoreType.DMA for _ in range(NBUF)],
            [pltpu.SemaphoreType.DMA for _ in range(NBUF)],
            [pltpu.SemaphoreType.DMA for _ in range(NBUF)],
        ],
    )
    def agg(x_hbm, ei_hbm, attr_hbm, part_hbm, degp_hbm,
            coli, rowi, attrf, ring, acc, dacc, msem, gsem, ssem, dsem):
        cid = lax.axis_index("c")
        sid = lax.axis_index("s")
        wid = sid * NC + cid

        def zfill(i, _):
            zero = jnp.zeros((L,), jnp.float32)
            for k in range(d // L):
                ring[0][i, pl.ds(k * L, L)] = zero
            return 0
        lax.fori_loop(0, C, zfill, 0)
        for k in range(C // L):
            attrf[0][0, pl.ds(k * L, L)] = jnp.zeros((L,), jnp.float32)
        for j in range(ZCOPIES):
            zoff = pl.multiple_of(sid * rows_per_tile + j * C, 8)
            pltpu.sync_copy(ring[0], acc.at[pl.ds(zoff, C)])
            pltpu.sync_copy(attrf[0].at[0], dacc.at[pl.ds(zoff, C)])
        plsc.subcore_barrier()

        def idx_load(g, m):
            pltpu.async_copy(ei_hbm.at[1, wid, g], coli[m], msem[m])
            pltpu.async_copy(ei_hbm.at[0, wid, g], rowi[m], msem[m])
            pltpu.async_copy(attr_hbm.at[wid, g], attrf[m], msem[m])

        def idx_wait(m):
            pltpu.make_async_copy(ei_hbm.at[1, wid, 0], coli[m], msem[m]).wait()
            pltpu.make_async_copy(ei_hbm.at[0, wid, 0], rowi[m], msem[m]).wait()
            pltpu.make_async_copy(attr_hbm.at[wid, 0], attrf[m], msem[m]).wait()

        def gather(b, m):
            pltpu.async_copy(x_hbm.at[coli[m].at[0]], ring[b], gsem[b])

        def gather_wait(b, m):
            pltpu.make_async_copy(x_hbm.at[coli[m].at[0]], ring[b], gsem[b]).wait()

        def scatter(b, m):
            pltpu.async_copy(ring[b], acc.at[rowi[m].at[0]], ssem[b], add=True)
            pltpu.async_copy(attrf[m].at[0], dacc.at[rowi[m].at[0]], dsem[b], add=True)

        def scatter_wait(b, m):
            pltpu.make_async_copy(ring[b], acc.at[rowi[m].at[0]], ssem[b]).wait()
            pltpu.make_async_copy(attrf[m].at[0], dacc.at[rowi[m].at[0]], dsem[b]).wait()

        def process(b, m):
            def scale(j16, _):
                av = attrf[m][0, pl.ds(j16 * L, L)]
                for j in range(L):
                    a = av[j]
                    base = j16 * L + j
                    for k in range(d // L):
                        sl = pl.ds(k * L, L)
                        ring[b][base, sl] = ring[b][base, sl] * a
                return 0
            lax.fori_loop(0, C // L, scale, 0)

        for g in range(NBUF):
            idx_load(g, g)
        for g in range(2):
            idx_wait(g)
            gather(g, g)

        def outer(o, _):
            for m in range(IB):
                g = o * IB + m
                b = m % NBUF

                @pl.when(g + 2 < lc)
                def _():
                    idx_wait((m + 2) % IB)

                @pl.when(g >= 2)
                def _():
                    scatter_wait((b + 2) % NBUF, (m + IB - 2) % IB)

                @pl.when(g + 2 < lc)
                def _():
                    gather((b + 2) % NBUF, (m + 2) % IB)

                gather_wait(b, m)
                process(b, m)
                scatter(b, m)

                @pl.when(g + NBUF < chunks)
                def _():
                    idx_load(g + NBUF, (m + NBUF) % IB)
            return 0
        lax.fori_loop(0, lc // IB, outer, 0)

        scatter_wait((lc - 2) % NBUF, (lc - 2) % IB)
        scatter_wait((lc - 1) % NBUF, (lc - 1) % IB)

        for g in range(lc + NBUF, chunks):
            idx_load(g, g % IB)
        for g in range(lc, chunks):
            b = g % NBUF
            m = g % IB
            idx_wait(m)
            gather(b, m)
            gather_wait(b, m)
            process(b, m)
            scatter(b, m)
            scatter_wait(b, m)

        plsc.subcore_barrier()
        woff = pl.multiple_of(sid * rows_per_tile, 8)
        pltpu.sync_copy(
            acc.at[pl.ds(woff, rows_per_tile)],
            part_hbm.at[cid, pl.ds(woff, rows_per_tile)],
        )
        pltpu.sync_copy(
            dacc.at[pl.ds(woff, rows_per_tile)],
            degp_hbm.at[cid, pl.ds(woff, rows_per_tile)],
        )

    return agg


def _finish_body(p_ref, dg_ref, w_ref, b_ref, o_ref):
    xp = p_ref[0] + p_ref[1]
    acc = lax.dot_general(
        xp, w_ref[...], (((1,), (1,)), ((), ())),
        preferred_element_type=jnp.float32,
    )
    dgs = dg_ref[0] + dg_ref[1]
    o_ref[...] = acc + dgs[:, None] * b_ref[...][None, :]


def _finish(p, dg, W, b, blk=2048):
    _, n_pad, d = p.shape
    grid = n_pad // blk
    return pl.pallas_call(
        _finish_body,
        grid=(grid,),
        in_specs=[
            pl.BlockSpec((NC, blk, d), lambda i: (0, i, 0)),
            pl.BlockSpec((NC, blk), lambda i: (0, i)),
            pl.BlockSpec((d, d), lambda i: (0, 0)),
            pl.BlockSpec((d,), lambda i: (0,)),
        ],
        out_specs=pl.BlockSpec((blk, d), lambda i: (i, 0)),
        out_shape=jax.ShapeDtypeStruct((n_pad, d), jnp.float32),
    )(p, dg, W, b)


def kernel(x, edge_index, edge_attr, W, b, W1, b1, W2, b2):
    n, d = x.shape
    e = edge_attr.shape[0]
    epw = e // NW
    chunks = epw // C
    ei = edge_index.astype(jnp.int32).reshape(2, NW, chunks, 1, C)
    attr = edge_attr.astype(jnp.float32).reshape(NW, chunks, 1, C)
    partials, degp = _make_agg(n, d, e)(x.astype(jnp.float32), ei, attr)
    return _finish(partials, degp, W, b)[:n]


# speedup vs baseline: 12.1415x; 1.0301x over previous
import functools

import jax
import jax.numpy as jnp
from jax import lax
from jax.experimental import pallas as pl
from jax.experimental.pallas import tpu as pltpu
from jax.experimental.pallas import tpu_sc as plsc

NC = 2
NS = 16
NW = NC * NS
L = 16

C = 80
NBUF = 4
ZCOPIES = 8


def _make_agg(n, d, e):
    epw = e // NW
    chunks = epw // C
    IB = 2 * NBUF
    lc = chunks - chunks % IB
    assert lc >= 2 * IB
    rows_per_tile = -(-n // (NS * C * ZCOPIES)) * C * ZCOPIES
    n_pad = rows_per_tile * NS

    @functools.partial(
        pl.kernel,
        out_type=(
            jax.ShapeDtypeStruct((NC, n_pad, d), jnp.float32),
            jax.ShapeDtypeStruct((NC, n_pad), jnp.float32),
        ),
        mesh=plsc.VectorSubcoreMesh(core_axis_name="c", subcore_axis_name="s"),
        scratch_types=[
            [pltpu.VMEM((1, C), jnp.int32) for _ in range(IB)],
            [pltpu.VMEM((1, C), jnp.int32) for _ in range(IB)],
            [pltpu.VMEM((1, C), jnp.float32) for _ in range(IB)],
            [pltpu.VMEM((C, d), jnp.float32) for _ in range(NBUF)],
            pltpu.VMEM_SHARED((n_pad, d), jnp.float32),
            pltpu.VMEM_SHARED((n_pad,), jnp.float32),
            [pltpu.SemaphoreType.DMA for _ in range(IB)],
            [pltpu.SemaphoreType.DMA for _ in range(NBUF)],
            [pltpu.SemaphoreType.DMA for _ in range(NBUF)],
            [pltpu.SemaphoreType.DMA for _ in range(NBUF)],
        ],
    )
    def agg(x_hbm, ei_hbm, attr_hbm, part_hbm, degp_hbm,
            coli, rowi, attrf, ring, acc, dacc, msem, gsem, ssem, dsem):
        cid = lax.axis_index("c")
        sid = lax.axis_index("s")
        wid = sid * NC + cid

        def zfill(i, _):
            zero = jnp.zeros((L,), jnp.float32)
            for k in range(d // L):
                ring[0][i, pl.ds(k * L, L)] = zero
            return 0
        lax.fori_loop(0, C, zfill, 0)
        for k in range(C // L):
            attrf[0][0, pl.ds(k * L, L)] = jnp.zeros((L,), jnp.float32)
        for j in range(ZCOPIES):
            zoff = pl.multiple_of(sid * rows_per_tile + j * C, 8)
            pltpu.sync_copy(ring[0], acc.at[pl.ds(zoff, C)])
            pltpu.sync_copy(attrf[0].at[0], dacc.at[pl.ds(zoff, C)])
        plsc.subcore_barrier()

        def idx_load(g, m):
            pltpu.async_copy(ei_hbm.at[1, wid, g], coli[m], msem[m])
            pltpu.async_copy(ei_hbm.at[0, wid, g], rowi[m], msem[m])
            pltpu.async_copy(attr_hbm.at[wid, g], attrf[m], msem[m])

        def idx_wait(m):
            pltpu.make_async_copy(ei_hbm.at[1, wid, 0], coli[m], msem[m]).wait()
            pltpu.make_async_copy(ei_hbm.at[0, wid, 0], rowi[m], msem[m]).wait()
            pltpu.make_async_copy(attr_hbm.at[wid, 0], attrf[m], msem[m]).wait()

        def gather(b, m):
            pltpu.async_copy(x_hbm.at[coli[m].at[0]], ring[b], gsem[b])

        def gather_wait(b, m):
            pltpu.make_async_copy(x_hbm.at[coli[m].at[0]], ring[b], gsem[b]).wait()

        def scatter(b, m):
            pltpu.async_copy(ring[b], acc.at[rowi[m].at[0]], ssem[b], add=True)
            pltpu.async_copy(attrf[m].at[0], dacc.at[rowi[m].at[0]], dsem[b], add=True)

        def scatter_wait(b, m):
            pltpu.make_async_copy(ring[b], acc.at[rowi[m].at[0]], ssem[b]).wait()
            pltpu.make_async_copy(attrf[m].at[0], dacc.at[rowi[m].at[0]], dsem[b]).wait()

        def process(b, m):
            def scale(j16, _):
                av = attrf[m][0, pl.ds(j16 * L, L)]
                for j in range(L):
                    a = av[j]
                    base = j16 * L + j
                    for k in range(d // L):
                        sl = pl.ds(k * L, L)
                        ring[b][base, sl] = ring[b][base, sl] * a
                return 0
            lax.fori_loop(0, C // L, scale, 0)

        for g in range(NBUF):
            idx_load(g, g)
        for g in range(2):
            idx_wait(g)
            gather(g, g)

        def outer(o, _):
            for m in range(IB):
                g = o * IB + m
                b = m % NBUF

                @pl.when(g + 2 < lc)
                def _():
                    idx_wait((m + 2) % IB)

                @pl.when(g >= 2)
                def _():
                    scatter_wait((b + 2) % NBUF, (m + IB - 2) % IB)

                @pl.when(g + 2 < lc)
                def _():
                    gather((b + 2) % NBUF, (m + 2) % IB)

                gather_wait(b, m)
                process(b, m)
                scatter(b, m)

                @pl.when(g + NBUF < chunks)
                def _():
                    idx_load(g + NBUF, (m + NBUF) % IB)
            return 0
        lax.fori_loop(0, lc // IB, outer, 0)

        scatter_wait((lc - 2) % NBUF, (lc - 2) % IB)
        scatter_wait((lc - 1) % NBUF, (lc - 1) % IB)

        for g in range(lc + NBUF, chunks):
            idx_load(g, g % IB)
        for g in range(lc, chunks):
            b = g % NBUF
            m = g % IB
            idx_wait(m)
            gather(b, m)
            gather_wait(b, m)
            process(b, m)
            scatter(b, m)
            scatter_wait(b, m)

        plsc.subcore_barrier()
        woff = pl.multiple_of(sid * rows_per_tile, 8)
        pltpu.sync_copy(
            acc.at[pl.ds(woff, rows_per_tile)],
            part_hbm.at[cid, pl.ds(woff, rows_per_tile)],
        )
        pltpu.sync_copy(
            dacc.at[pl.ds(woff, rows_per_tile)],
            degp_hbm.at[cid, pl.ds(woff, rows_per_tile)],
        )

    return agg


def _finish_body(p_ref, dg_ref, w_ref, b_ref, o_ref):
    xp = p_ref[0] + p_ref[1]
    acc = lax.dot_general(
        xp, w_ref[...], (((1,), (1,)), ((), ())),
        preferred_element_type=jnp.float32,
    )
    dgs = dg_ref[0] + dg_ref[1]
    o_ref[...] = acc + dgs[:, None] * b_ref[...][None, :]


def _finish(p, dg, W, b, n, blk=2048):
    _, n_pad, d = p.shape
    grid = n_pad // blk
    return pl.pallas_call(
        _finish_body,
        grid=(grid,),
        in_specs=[
            pl.BlockSpec((NC, blk, d), lambda i: (0, i, 0)),
            pl.BlockSpec((NC, blk), lambda i: (0, i)),
            pl.BlockSpec((d, d), lambda i: (0, 0)),
            pl.BlockSpec((d,), lambda i: (0,)),
        ],
        out_specs=pl.BlockSpec((blk, d), lambda i: (i, 0)),
        out_shape=jax.ShapeDtypeStruct((n, d), jnp.float32),
    )(p, dg, W, b)


def kernel(x, edge_index, edge_attr, W, b, W1, b1, W2, b2):
    n, d = x.shape
    e = edge_attr.shape[0]
    epw = e // NW
    chunks = epw // C
    ei = edge_index.astype(jnp.int32).reshape(2, NW, chunks, 1, C)
    attr = edge_attr.astype(jnp.float32).reshape(NW, chunks, 1, C)
    partials, degp = _make_agg(n, d, e)(x.astype(jnp.float32), ei, attr)
    return _finish(partials, degp, W, b, n)


# speedup vs baseline: 12.5503x; 1.0337x over previous
import functools

import jax
import jax.numpy as jnp
from jax import lax
from jax.experimental import pallas as pl
from jax.experimental.pallas import tpu as pltpu
from jax.experimental.pallas import tpu_sc as plsc

NC = 2
NS = 16
NW = NC * NS
L = 16

C = 80
NBUF = 4
ZCOPIES = 8


GC = 5


def _make_agg(n, d, e):
    epw = e // NW
    chunks = epw // C
    ngroups = chunks // GC
    BLKS = 4 * GC
    lc = chunks - chunks % BLKS
    assert chunks % GC == 0 and lc >= 2 * BLKS
    rows_per_tile = -(-n // (NS * C * ZCOPIES)) * C * ZCOPIES
    n_pad = rows_per_tile * NS

    @functools.partial(
        pl.kernel,
        out_type=(
            jax.ShapeDtypeStruct((NC, n_pad, d), jnp.float32),
            jax.ShapeDtypeStruct((NC, n_pad), jnp.float32),
        ),
        mesh=plsc.VectorSubcoreMesh(core_axis_name="c", subcore_axis_name="s"),
        scratch_types=[
            [pltpu.VMEM((GC, C), jnp.int32) for _ in range(2)],
            [pltpu.VMEM((GC, C), jnp.int32) for _ in range(2)],
            [pltpu.VMEM((GC, C), jnp.float32) for _ in range(2)],
            [pltpu.VMEM((C, d), jnp.float32) for _ in range(NBUF)],
            pltpu.VMEM_SHARED((n_pad, d), jnp.float32),
            pltpu.VMEM_SHARED((n_pad,), jnp.float32),
            [pltpu.SemaphoreType.DMA for _ in range(2)],
            [pltpu.SemaphoreType.DMA for _ in range(NBUF)],
            [pltpu.SemaphoreType.DMA for _ in range(NBUF)],
            [pltpu.SemaphoreType.DMA for _ in range(NBUF)],
        ],
    )
    def agg(x_hbm, ei_hbm, attr_hbm, part_hbm, degp_hbm,
            coli, rowi, attrf, ring, acc, dacc, msem, gsem, ssem, dsem):
        cid = lax.axis_index("c")
        sid = lax.axis_index("s")
        wid = sid * NC + cid

        def zfill(i, _):
            zero = jnp.zeros((L,), jnp.float32)
            for k in range(d // L):
                ring[0][i, pl.ds(k * L, L)] = zero
            return 0
        lax.fori_loop(0, C, zfill, 0)
        for k in range(C // L):
            attrf[0][0, pl.ds(k * L, L)] = jnp.zeros((L,), jnp.float32)
        for j in range(ZCOPIES):
            zoff = pl.multiple_of(sid * rows_per_tile + j * C, 8)
            pltpu.sync_copy(ring[0], acc.at[pl.ds(zoff, C)])
            pltpu.sync_copy(attrf[0].at[0], dacc.at[pl.ds(zoff, C)])
        plsc.subcore_barrier()

        def idx_load(G, gb):
            pltpu.async_copy(ei_hbm.at[1, wid, G], coli[gb], msem[gb])
            pltpu.async_copy(ei_hbm.at[0, wid, G], rowi[gb], msem[gb])
            pltpu.async_copy(attr_hbm.at[wid, G], attrf[gb], msem[gb])

        def idx_wait(gb):
            pltpu.make_async_copy(ei_hbm.at[1, wid, 0], coli[gb], msem[gb]).wait()
            pltpu.make_async_copy(ei_hbm.at[0, wid, 0], rowi[gb], msem[gb]).wait()
            pltpu.make_async_copy(attr_hbm.at[wid, 0], attrf[gb], msem[gb]).wait()

        def gather(b, gb, r):
            pltpu.async_copy(x_hbm.at[coli[gb].at[r]], ring[b], gsem[b])

        def gather_wait(b, gb, r):
            pltpu.make_async_copy(x_hbm.at[coli[gb].at[r]], ring[b],
                                  gsem[b]).wait()

        def scatter(b, gb, r):
            pltpu.async_copy(ring[b], acc.at[rowi[gb].at[r]], ssem[b], add=True)
            pltpu.async_copy(attrf[gb].at[r], dacc.at[rowi[gb].at[r]],
                             dsem[b], add=True)

        def scatter_wait(b, gb, r):
            pltpu.make_async_copy(ring[b], acc.at[rowi[gb].at[r]],
                                  ssem[b]).wait()
            pltpu.make_async_copy(attrf[gb].at[r], dacc.at[rowi[gb].at[r]],
                                  dsem[b]).wait()

        def process(b, gb, r):
            def scale(j16, _):
                av = attrf[gb][r, pl.ds(j16 * L, L)]
                for j in range(L):
                    a = av[j]
                    base = j16 * L + j
                    for k in range(d // L):
                        sl = pl.ds(k * L, L)
                        ring[b][base, sl] = ring[b][base, sl] * a
                return 0
            lax.fori_loop(0, C // L, scale, 0)

        idx_load(0, 0)
        idx_load(1, 1)
        idx_wait(0)
        gather(0, 0, 0)
        gather(1, 0, 1)

        def outer(o, _):
            for s in range(BLKS):
                g = o * BLKS + s
                G = o * (BLKS // GC) + s // GC
                gb = (s // GC) % 2
                r = s % GC
                b = s % NBUF

                if r == GC - 2:
                    @pl.when(g + 2 < lc)
                    def _():
                        idx_wait((gb + 1) % 2)

                @pl.when(g >= 2)
                def _():
                    gb2 = gb if r >= 2 else (gb + 1) % 2
                    r2 = r - 2 if r >= 2 else r + GC - 2
                    scatter_wait((b + 2) % NBUF, gb2, r2)

                @pl.when(g + 2 < lc)
                def _():
                    gb3 = gb if r < GC - 2 else (gb + 1) % 2
                    gather((b + 2) % NBUF, gb3, (r + 2) % GC)

                gather_wait(b, gb, r)
                process(b, gb, r)
                scatter(b, gb, r)

                if r == 1:
                    @pl.when(G >= 1)
                    def _():
                        idx_load(G + 1, (gb + 1) % 2)
            return 0
        lax.fori_loop(0, lc // BLKS, outer, 0)

        for g in (lc - 2, lc - 1):
            scatter_wait(g % NBUF, (g // GC) % 2, g % GC)

        idx_wait((lc // GC) % 2)
        for g in range(lc, chunks):
            b = g % NBUF
            gb = (g // GC) % 2
            r = g % GC
            gather(b, gb, r)
            gather_wait(b, gb, r)
            process(b, gb, r)
            scatter(b, gb, r)
            scatter_wait(b, gb, r)

        plsc.subcore_barrier()
        woff = pl.multiple_of(sid * rows_per_tile, 8)
        pltpu.sync_copy(
            acc.at[pl.ds(woff, rows_per_tile)],
            part_hbm.at[cid, pl.ds(woff, rows_per_tile)],
        )
        pltpu.sync_copy(
            dacc.at[pl.ds(woff, rows_per_tile)],
            degp_hbm.at[cid, pl.ds(woff, rows_per_tile)],
        )

    return agg


def _finish_body(p_ref, dg_ref, w_ref, b_ref, o_ref):
    xp = p_ref[0] + p_ref[1]
    acc = lax.dot_general(
        xp, w_ref[...], (((1,), (1,)), ((), ())),
        preferred_element_type=jnp.float32,
    )
    dgs = dg_ref[0] + dg_ref[1]
    o_ref[...] = acc + dgs[:, None] * b_ref[...][None, :]


def _finish(p, dg, W, b, n, blk=2048):
    _, n_pad, d = p.shape
    grid = n_pad // blk
    return pl.pallas_call(
        _finish_body,
        grid=(grid,),
        in_specs=[
            pl.BlockSpec((NC, blk, d), lambda i: (0, i, 0)),
            pl.BlockSpec((NC, blk), lambda i: (0, i)),
            pl.BlockSpec((d, d), lambda i: (0, 0)),
            pl.BlockSpec((d,), lambda i: (0,)),
        ],
        out_specs=pl.BlockSpec((blk, d), lambda i: (i, 0)),
        out_shape=jax.ShapeDtypeStruct((n, d), jnp.float32),
    )(p, dg, W, b)


def kernel(x, edge_index, edge_attr, W, b, W1, b1, W2, b2):
    n, d = x.shape
    e = edge_attr.shape[0]
    epw = e // NW
    chunks = epw // C
    ei = edge_index.astype(jnp.int32).reshape(2, NW, chunks // GC, GC, C)
    attr = edge_attr.astype(jnp.float32).reshape(NW, chunks // GC, GC, C)
    partials, degp = _make_agg(n, d, e)(x.astype(jnp.float32), ei, attr)
    return _finish(partials, degp, W, b, n)
